# baseline (speedup 1.0000x reference)
import functools

import jax
import jax.numpy as jnp
from jax import lax
from jax.experimental import pallas as pl
from jax.experimental.pallas import tpu as pltpu
from jax.experimental.pallas import tpu_sc as plsc

_NC = 2
_NS = 16
_NW = _NC * _NS
_CHUNK = 128


def _sc_gather(user_id, movie_id, user_emb, movie_emb):
    B = user_id.shape[0]
    D = user_emb.shape[1]
    b_per_w = B // _NW
    n_chunks = b_per_w // _CHUNK
    rows_blocks = B // _CHUNK

    uid2d = user_id.reshape(rows_blocks, _CHUNK)
    mid2d = movie_id.reshape(rows_blocks, _CHUNK)

    mesh = plsc.VectorSubcoreMesh(core_axis_name="c", subcore_axis_name="s")

    @functools.partial(
        pl.kernel,
        mesh=mesh,
        compiler_params=pltpu.CompilerParams(use_tc_tiling_on_sc=False),
        out_type=(
            jax.ShapeDtypeStruct((rows_blocks, _CHUNK, D), jnp.float32),
            jax.ShapeDtypeStruct((rows_blocks, _CHUNK, D), jnp.float32),
        ),
        scratch_types=[
            pltpu.VMEM((n_chunks, _CHUNK), jnp.int32),
            pltpu.VMEM((n_chunks, _CHUNK, D), jnp.float32),
            pltpu.VMEM((n_chunks, _CHUNK), jnp.int32),
            pltpu.VMEM((n_chunks, _CHUNK, D), jnp.float32),
            pltpu.SemaphoreType.DMA,
        ],
    )
    def gather_kernel(uid_hbm, mid_hbm, uemb_hbm, memb_hbm,
                      uf_hbm, mf_hbm, uidx_v, urows_v, midx_v, mrows_v, sem):
        wid = lax.axis_index("s") * _NC + lax.axis_index("c")
        blk = wid * n_chunks
        pltpu.sync_copy(uid_hbm.at[pl.ds(blk, n_chunks)], uidx_v)
        pltpu.sync_copy(mid_hbm.at[pl.ds(blk, n_chunks)], midx_v)
        copies = []
        for j in range(n_chunks):
            copies.append(
                pltpu.async_copy(uemb_hbm.at[uidx_v.at[j]], urows_v.at[j], sem))
            copies.append(
                pltpu.async_copy(memb_hbm.at[midx_v.at[j]], mrows_v.at[j], sem))
        for c in copies:
            c.wait()
        pltpu.sync_copy(urows_v, uf_hbm.at[pl.ds(blk, n_chunks)])
        pltpu.sync_copy(mrows_v, mf_hbm.at[pl.ds(blk, n_chunks)])

    uf, mf = gather_kernel(uid2d, mid2d, user_emb, movie_emb)
    return uf.reshape(B, D), mf.reshape(B, D)


def _mlp_body(uf_ref, mf_ref, w1a_ref, w1b_ref, b1_ref, w2_ref, b2_ref,
              w3_ref, b3_ref, w4_ref, b4_ref, out_ref):
    x = jnp.dot(uf_ref[...], w1a_ref[...], preferred_element_type=jnp.float32)
    x = x + jnp.dot(mf_ref[...], w1b_ref[...], preferred_element_type=jnp.float32)
    x = jnp.maximum(x + b1_ref[...], 0.0)
    x = jnp.dot(x, w2_ref[...], preferred_element_type=jnp.float32)
    x = jnp.maximum(x + b2_ref[...], 0.0)
    x = jnp.dot(x, w3_ref[...], preferred_element_type=jnp.float32)
    x = jnp.maximum(x + b3_ref[...], 0.0)
    x = jnp.dot(x, w4_ref[...], preferred_element_type=jnp.float32)
    y = jax.nn.sigmoid(x + b4_ref[...])
    out_ref[...] = y * 5.0 + 1.0


def _tc_mlp(uf, mf, W1, b1, W2, b2, W3, b3, W4, b4):
    B = uf.shape[0]
    D = uf.shape[1]
    W1a = W1[:D]
    W1b = W1[D:]
    grid = 8
    rows = B // grid

    def rb(r, c):
        def im(i):
            return (i, 0)
        return pl.BlockSpec((r, c), im)

    def full(a):
        return pl.BlockSpec(a.shape, lambda i: (0,) * a.ndim)

    b1r = b1.reshape(1, -1)
    b2r = b2.reshape(1, -1)
    b3r = b3.reshape(1, -1)
    b4r = b4.reshape(1, -1)

    return pl.pallas_call(
        _mlp_body,
        grid=(grid,),
        in_specs=[
            rb(rows, D), rb(rows, D),
            full(W1a), full(W1b), full(b1r),
            full(W2), full(b2r),
            full(W3), full(b3r),
            full(W4), full(b4r),
        ],
        out_specs=rb(rows, 1),
        out_shape=jax.ShapeDtypeStruct((B, 1), jnp.float32),
    )(uf, mf, W1a, W1b, b1r, W2, b2r, W3, b3r, W4, b4r)


def kernel(user_id, movie_id, user_emb, movie_emb, W1, b1, W2, b2, W3, b3, W4, b4):
    uf, mf = _sc_gather(user_id.astype(jnp.int32), movie_id.astype(jnp.int32),
                        user_emb, movie_emb)
    return _tc_mlp(uf, mf, W1, b1, W2, b2, W3, b3, W4, b4)


# speedup vs baseline: 3.3012x; 3.3012x over previous
import functools

import jax
import jax.numpy as jnp
from jax import lax
from jax.experimental import pallas as pl
from jax.experimental.pallas import tpu as pltpu
from jax.experimental.pallas import tpu_sc as plsc

_NC = 2
_NS = 16
_NW = _NC * _NS
_CHUNK = 128
_G = 8


def _sc_gather(user_id, movie_id, uembT, membT):
    B = user_id.shape[0]
    D = uembT.shape[0]
    b_per_w = B // _NW
    n_chunks = b_per_w // _CHUNK
    n_groups = _CHUNK // _G

    uid3 = user_id.reshape(_NW, n_chunks, _CHUNK)
    mid3 = movie_id.reshape(_NW, n_chunks, _CHUNK)

    mesh = plsc.VectorSubcoreMesh(core_axis_name="c", subcore_axis_name="s")

    @functools.partial(
        pl.kernel,
        mesh=mesh,
        compiler_params=pltpu.CompilerParams(needs_layout_passes=False),
        out_type=jax.ShapeDtypeStruct((B, 2 * D), jnp.float32),
        scratch_types=[
            pltpu.VMEM((n_chunks, _CHUNK), jnp.int32),
            pltpu.VMEM((n_chunks, _CHUNK), jnp.int32),
            pltpu.VMEM((_G, D, 128), jnp.float32),
            pltpu.VMEM((_G, D, 128), jnp.float32),
            pltpu.VMEM((_CHUNK, 2 * D), jnp.float32),
            pltpu.SemaphoreType.DMA,
        ],
    )
    def gather_kernel(uid_hbm, mid_hbm, ut_hbm, mt_hbm, out_hbm,
                      uidx_v, midx_v, uslab, mslab, rows_v, sem):
        wid = lax.axis_index("s") * _NC + lax.axis_index("c")
        pltpu.sync_copy(uid_hbm.at[wid], uidx_v)
        pltpu.sync_copy(mid_hbm.at[wid], midx_v)
        lanes = lax.iota(jnp.int32, 16)

        def extract(slab, k, lane, dst_row, lane0):
            li = jnp.full((16,), lane, jnp.int32)
            ks = jnp.full((16,), k, jnp.int32)
            lo = plsc.load_gather(slab, [ks, lanes, li])
            hi = plsc.load_gather(slab, [ks, lanes + 16, li])
            rows_v[dst_row, pl.ds(lane0, 16)] = lo
            rows_v[dst_row, pl.ds(lane0 + 16, 16)] = hi

        def per_group(c, g, _):
            seg = (g // 2) * 16
            idx16u = uidx_v[c, pl.ds(seg, 16)]
            idx16m = midx_v[c, pl.ds(seg, 16)]
            half = (g % 2) * 8
            vus = []
            vms = []
            copies = []
            for k in range(_G):
                sel = lanes == (half + k)
                vu = jnp.max(jnp.where(sel, idx16u, 0))
                vm = jnp.max(jnp.where(sel, idx16m, 0))
                vus.append(vu)
                vms.append(vm)
                copies.append(pltpu.async_copy(
                    ut_hbm.at[:, pl.ds((vu >> 7) * 128, 128)], uslab.at[k], sem))
                copies.append(pltpu.async_copy(
                    mt_hbm.at[:, pl.ds((vm >> 7) * 128, 128)], mslab.at[k], sem))
            for cp in copies:
                cp.wait()
            for k in range(_G):
                j = g * _G + k
                extract(uslab, k, vus[k] & 127, j, 0)
                extract(mslab, k, vms[k] & 127, j, D)
            return _

        for c in range(n_chunks):
            lax.fori_loop(0, n_groups, functools.partial(per_group, c), 0)
            pltpu.sync_copy(
                rows_v, out_hbm.at[pl.ds(wid * b_per_w + c * _CHUNK, _CHUNK)])

    return gather_kernel(uid3, mid3, uembT, membT)


def _mlp_body(x_ref, w1_ref, b1_ref, w2_ref, b2_ref,
              w3_ref, b3_ref, w4_ref, b4_ref, out_ref):
    x = jnp.dot(x_ref[...], w1_ref[...], preferred_element_type=jnp.float32)
    x = jnp.maximum(x + b1_ref[...], 0.0)
    x = jnp.dot(x, w2_ref[...], preferred_element_type=jnp.float32)
    x = jnp.maximum(x + b2_ref[...], 0.0)
    x = jnp.dot(x, w3_ref[...], preferred_element_type=jnp.float32)
    x = jnp.maximum(x + b3_ref[...], 0.0)
    x = jnp.dot(x, w4_ref[...], preferred_element_type=jnp.float32)
    y = jax.nn.sigmoid(x + b4_ref[...])
    out_ref[...] = y * 5.0 + 1.0


def _tc_mlp(x, W1, b1, W2, b2, W3, b3, W4, b4):
    B = x.shape[0]
    K = x.shape[1]
    grid = 8
    rows = B // grid

    def rb(r, c):
        return pl.BlockSpec((r, c), lambda i: (i, 0))

    def full(a):
        return pl.BlockSpec(a.shape, lambda i: (0,) * a.ndim)

    b1r = b1.reshape(1, -1)
    b2r = b2.reshape(1, -1)
    b3r = b3.reshape(1, -1)
    b4r = b4.reshape(1, -1)

    return pl.pallas_call(
        _mlp_body,
        grid=(grid,),
        in_specs=[
            rb(rows, K),
            full(W1), full(b1r),
            full(W2), full(b2r),
            full(W3), full(b3r),
            full(W4), full(b4r),
        ],
        out_specs=rb(rows, 1),
        out_shape=jax.ShapeDtypeStruct((B, 1), jnp.float32),
    )(x, W1, b1r, W2, b2r, W3, b3r, W4, b4r)


def kernel(user_id, movie_id, user_emb, movie_emb, W1, b1, W2, b2, W3, b3, W4, b4):
    x = _sc_gather(user_id.astype(jnp.int32), movie_id.astype(jnp.int32),
                   user_emb.T, movie_emb.T)
    return _tc_mlp(x, W1, b1, W2, b2, W3, b3, W4, b4)


# speedup vs baseline: 3.3834x; 1.0249x over previous
import functools

import jax
import jax.numpy as jnp
from jax import lax
from jax.experimental import pallas as pl
from jax.experimental.pallas import tpu as pltpu
from jax.experimental.pallas import tpu_sc as plsc

_NC = 2
_NS = 16
_NW = _NC * _NS
_CHUNK = 128
_G = 4


def _sc_gather(user_id, movie_id, uembT, membT):
    B = user_id.shape[0]
    D = uembT.shape[0]
    b_per_w = B // _NW
    n_chunks = b_per_w // _CHUNK
    n_groups = _CHUNK // _G

    uid3 = user_id.reshape(_NW, n_chunks, _CHUNK)
    mid3 = movie_id.reshape(_NW, n_chunks, _CHUNK)

    mesh = plsc.VectorSubcoreMesh(core_axis_name="c", subcore_axis_name="s")

    @functools.partial(
        pl.kernel,
        mesh=mesh,
        compiler_params=pltpu.CompilerParams(needs_layout_passes=False),
        out_type=jax.ShapeDtypeStruct((B // 2, 4 * D), jnp.float32),
        scratch_types=[
            pltpu.VMEM((n_chunks, _CHUNK), jnp.int32),
            pltpu.VMEM((n_chunks, _CHUNK), jnp.int32),
            pltpu.VMEM((2, _G, D, 128), jnp.float32),
            pltpu.VMEM((2, _G, D, 128), jnp.float32),
            pltpu.VMEM((_CHUNK // 2, 4 * D), jnp.float32),
            pltpu.SemaphoreType.DMA,
            pltpu.SemaphoreType.DMA,
        ],
    )
    def gather_kernel(uid_hbm, mid_hbm, ut_hbm, mt_hbm, out_hbm,
                      uidx_v, midx_v, uslab, mslab, rows_v, sem0, sem1):
        wid = lax.axis_index("s") * _NC + lax.axis_index("c")
        pltpu.sync_copy(uid_hbm.at[wid], uidx_v)
        pltpu.sync_copy(mid_hbm.at[wid], midx_v)
        lanes = lax.iota(jnp.int32, 16)
        sems = (sem0, sem1)

        def fire(c_, g, p, sem):
            seg = (g // 4) * 16
            off = (g % 4) * 4
            idx16u = uidx_v[c_, pl.ds(seg, 16)]
            idx16m = midx_v[c_, pl.ds(seg, 16)]
            vus, vms = [], []
            for k in range(_G):
                sel = lanes == (off + k)
                vu = jnp.max(jnp.where(sel, idx16u, 0))
                vm = jnp.max(jnp.where(sel, idx16m, 0))
                vus.append(vu)
                vms.append(vm)
                pltpu.async_copy(
                    ut_hbm.at[:, pl.ds((vu >> 7) * 128, 128)],
                    uslab.at[p, k], sem)
                pltpu.async_copy(
                    mt_hbm.at[:, pl.ds((vm >> 7) * 128, 128)],
                    mslab.at[p, k], sem)
            return tuple(vus) + tuple(vms)

        def drain_extract(g, p, sem, ids):
            for k in range(_G):
                pltpu.make_async_copy(
                    ut_hbm.at[:, pl.ds(0, 128)], uslab.at[p, k], sem).wait()
                pltpu.make_async_copy(
                    mt_hbm.at[:, pl.ds(0, 128)], mslab.at[p, k], sem).wait()
            for k in range(_G):
                j2 = g * (_G // 2) + k // 2
                half = (k % 2) * 2 * D
                for slab, v, lane0 in ((uslab, ids[k], half),
                                       (mslab, ids[_G + k], half + D)):
                    li = jnp.full((16,), v & 127, jnp.int32)
                    ks = jnp.full((16,), k, jnp.int32)
                    sl = slab.at[p]
                    lo = plsc.load_gather(sl, [ks, lanes, li])
                    hi = plsc.load_gather(sl, [ks, lanes + 16, li])
                    rows_v[j2, pl.ds(lane0, 16)] = lo
                    rows_v[j2, pl.ds(lane0 + 16, 16)] = hi

        n_half = n_groups // 2

        for c in range(n_chunks):
            def pair_body(i, ids0, c=c):
                g0 = 2 * i
                ids1 = fire(c, g0 + 1, 1, sem1)
                drain_extract(g0, 0, sem0, ids0)
                ids0n = fire(c, g0 + 2, 0, sem0)
                drain_extract(g0 + 1, 1, sem1, ids1)
                return ids0n

            ids0 = fire(c, 0, 0, sem0)
            ids0 = lax.fori_loop(0, n_half - 1, pair_body, ids0)
            g0 = 2 * (n_half - 1)
            ids1 = fire(c, g0 + 1, 1, sem1)
            drain_extract(g0, 0, sem0, ids0)
            drain_extract(g0 + 1, 1, sem1, ids1)
            pltpu.sync_copy(
                rows_v,
                out_hbm.at[pl.ds(wid * (b_per_w // 2) + c * (_CHUNK // 2),
                                 _CHUNK // 2)])

    return gather_kernel(uid3, mid3, uembT, membT)


def _mlp_body(x_ref, w1_ref, b1_ref, w2_ref, b2_ref,
              w3_ref, b3_ref, w4_ref, b4_ref, out_ref):
    def chain(x):
        h = jnp.dot(x, w1_ref[...], preferred_element_type=jnp.float32)
        h = jnp.maximum(h + b1_ref[...], 0.0)
        h = jnp.dot(h, w2_ref[...], preferred_element_type=jnp.float32)
        h = jnp.maximum(h + b2_ref[...], 0.0)
        h = jnp.dot(h, w3_ref[...], preferred_element_type=jnp.float32)
        h = jnp.maximum(h + b3_ref[...], 0.0)
        h = jnp.dot(h, w4_ref[...], preferred_element_type=jnp.float32)
        return jax.nn.sigmoid(h + b4_ref[...]) * 5.0 + 1.0

    x2 = x_ref[...]
    ya = chain(x2[:, 0:64])
    yb = chain(x2[:, 64:128])
    out_ref[...] = jnp.concatenate([ya, yb], axis=1)


def _tc_mlp(x2, W1, b1, W2, b2, W3, b3, W4, b4):
    B2 = x2.shape[0]
    grid = 8
    rows = B2 // grid

    def full(a):
        return pl.BlockSpec(a.shape, lambda i: (0,) * a.ndim)

    b1r = b1.reshape(1, -1)
    b2r = b2.reshape(1, -1)
    b3r = b3.reshape(1, -1)
    b4r = b4.reshape(1, -1)

    out = pl.pallas_call(
        _mlp_body,
        grid=(grid,),
        in_specs=[
            pl.BlockSpec((rows, 128), lambda i: (i, 0)),
            full(W1), full(b1r),
            full(W2), full(b2r),
            full(W3), full(b3r),
            full(W4), full(b4r),
        ],
        out_specs=pl.BlockSpec((rows, 2), lambda i: (i, 0)),
        out_shape=jax.ShapeDtypeStruct((B2, 2), jnp.float32),
    )(x2, W1, b1r, W2, b2r, W3, b3r, W4, b4r)
    return out.reshape(2 * B2, 1)


def kernel(user_id, movie_id, user_emb, movie_emb, W1, b1, W2, b2, W3, b3, W4, b4):
    x2 = _sc_gather(user_id.astype(jnp.int32), movie_id.astype(jnp.int32),
                    user_emb.T, movie_emb.T)
    return _tc_mlp(x2, W1, b1, W2, b2, W3, b3, W4, b4)
